# Initial kernel scaffold; baseline (speedup 1.0000x reference)
import jax
import jax.numpy as jnp
from jax.experimental import pallas as pl


def kernel(x, edge_index, batch, W1, b1, W2, b2, W3, b3, lin_W, lin_b):
    raise NotImplementedError("write your pallas kernel here")


# speedup vs baseline: 6.6841x; 6.6841x over previous
import functools

import jax
import jax.numpy as jnp
from jax import lax
from jax.experimental import pallas as pl
from jax.experimental.pallas import tpu as pltpu
from jax.experimental.pallas import tpu_sc as plsc

N = 10000
E = 320000
D = 128
NUM_GRAPHS = 128

NTILES = 32
CHUNK = 128
CPT = 80
EPT = CHUNK * CPT
E_PAD = EPT * NTILES
N_PAD = 10240
RPT = N_PAD // 16

BLK = 512


def _mesh():
    return plsc.VectorSubcoreMesh(core_axis_name="c", subcore_axis_name="s")


def _deg_body(dst_hbm, zeros_hbm, ones_hbm, out_hbm, dst_v, ones_v, acc_sh):
    c = lax.axis_index("c")
    s = lax.axis_index("s")
    wid = c * 16 + s
    pltpu.sync_copy(zeros_hbm, acc_sh.at[pl.ds(s * RPT, RPT)])
    pltpu.sync_copy(dst_hbm.at[wid], dst_v)
    pltpu.sync_copy(ones_hbm, ones_v)
    plsc.subcore_barrier()

    def step(i, carry):
        pltpu.sync_copy(ones_v, acc_sh.at[dst_v.at[i]], add=True)
        return carry

    lax.fori_loop(0, CPT, step, 0)
    plsc.subcore_barrier()
    pltpu.sync_copy(acc_sh.at[pl.ds(s * RPT, RPT)],
                    out_hbm.at[c, pl.ds(s * RPT, RPT)])


@jax.jit
def _deg_call(dst_p, zeros128, ones128):
    return pl.kernel(
        _deg_body,
        out_type=jax.ShapeDtypeStruct((2, N_PAD, D), jnp.float32),
        mesh=_mesh(),
        scratch_types=[
            pltpu.VMEM((CPT, CHUNK), jnp.int32),
            pltpu.VMEM((CHUNK, D), jnp.float32),
            pltpu.VMEM_SHARED((N_PAD, D), jnp.float32),
        ],
    )(dst_p, zeros128, ones128)


def _spmm_body(x_hbm, src_hbm, dst_hbm, zeros_hbm, out_hbm,
               src_v, dst_v, rows_v, acc_sh, sem):
    c = lax.axis_index("c")
    s = lax.axis_index("s")
    wid = c * 16 + s
    pltpu.sync_copy(zeros_hbm, acc_sh.at[pl.ds(s * RPT, RPT)])
    pltpu.sync_copy(src_hbm.at[wid], src_v)
    pltpu.sync_copy(dst_hbm.at[wid], dst_v)
    plsc.subcore_barrier()

    def step(i, carry):
        pltpu.async_copy(x_hbm.at[src_v.at[i]], rows_v, sem).wait()
        pltpu.sync_copy(rows_v, acc_sh.at[dst_v.at[i]], add=True)
        return carry

    lax.fori_loop(0, CPT, step, 0)
    plsc.subcore_barrier()
    pltpu.sync_copy(acc_sh.at[pl.ds(s * RPT, RPT)],
                    out_hbm.at[c, pl.ds(s * RPT, RPT)])


@jax.jit
def _spmm_call(x_pad, src_p, dst_p, zeros128):
    return pl.kernel(
        _spmm_body,
        out_type=jax.ShapeDtypeStruct((2, N_PAD, D), jnp.float32),
        mesh=_mesh(),
        scratch_types=[
            pltpu.VMEM((CPT, CHUNK), jnp.int32),
            pltpu.VMEM((CPT, CHUNK), jnp.int32),
            pltpu.VMEM((CHUNK, D), jnp.float32),
            pltpu.VMEM_SHARED((N_PAD, D), jnp.float32),
            pltpu.SemaphoreType.DMA,
        ],
    )(x_pad, src_p, dst_p, zeros128)


def _dense1_body(x_ref, w_ref, deg_ref, xs_ref, dis_ref):
    deg = deg_ref[...]
    dis = jnp.where(deg > 0, lax.rsqrt(deg), 0.0)
    h = jnp.dot(x_ref[...], w_ref[...], preferred_element_type=jnp.float32)
    xs_ref[...] = h * dis[:, None]
    dis_ref[...] = dis


@jax.jit
def _dense1_call(x_pad, W1, deg_pad):
    return pl.pallas_call(
        _dense1_body,
        grid=(N_PAD // BLK,),
        in_specs=[
            pl.BlockSpec((BLK, D), lambda i: (i, 0)),
            pl.BlockSpec((D, D), lambda i: (0, 0)),
            pl.BlockSpec((BLK,), lambda i: (i,)),
        ],
        out_specs=[
            pl.BlockSpec((BLK, D), lambda i: (i, 0)),
            pl.BlockSpec((BLK,), lambda i: (i,)),
        ],
        out_shape=[
            jax.ShapeDtypeStruct((N_PAD, D), jnp.float32),
            jax.ShapeDtypeStruct((N_PAD,), jnp.float32),
        ],
    )(x_pad, W1, deg_pad)


def _mid_body(ya_ref, yb_ref, xp_ref, dis_ref, b_ref, w_ref, out_ref):
    dis = dis_ref[...]
    t = (ya_ref[...] + yb_ref[...] + xp_ref[...]) * dis[:, None] + b_ref[...]
    h = jnp.maximum(t, 0.0)
    out_ref[...] = jnp.dot(h, w_ref[...],
                           preferred_element_type=jnp.float32) * dis[:, None]


@jax.jit
def _mid_call(ya, yb, xp, dis, b2d, W):
    return pl.pallas_call(
        _mid_body,
        grid=(N_PAD // BLK,),
        in_specs=[
            pl.BlockSpec((BLK, D), lambda i: (i, 0)),
            pl.BlockSpec((BLK, D), lambda i: (i, 0)),
            pl.BlockSpec((BLK, D), lambda i: (i, 0)),
            pl.BlockSpec((BLK,), lambda i: (i,)),
            pl.BlockSpec((1, D), lambda i: (0, 0)),
            pl.BlockSpec((D, D), lambda i: (0, 0)),
        ],
        out_specs=pl.BlockSpec((BLK, D), lambda i: (i, 0)),
        out_shape=jax.ShapeDtypeStruct((N_PAD, D), jnp.float32),
    )(ya, yb, xp, dis, b2d, W)


def _final_body(ya_ref, yb_ref, xp_ref, dis_ref, b_ref, batch_ref,
                linw_ref, linb_ref, out_ref, sums, cnts):
    i = pl.program_id(0)

    @pl.when(i == 0)
    def _():
        sums[...] = jnp.zeros_like(sums)
        cnts[...] = jnp.zeros_like(cnts)

    dis = dis_ref[...]
    t = (ya_ref[...] + yb_ref[...] + xp_ref[...]) * dis[:, None] + b_ref[...]
    h = jnp.maximum(t, 0.0)
    bt = batch_ref[...]
    onehot = (lax.broadcasted_iota(jnp.int32, (BLK, NUM_GRAPHS), 1)
              == bt[:, None]).astype(jnp.float32)
    dn = (((0,), (0,)), ((), ()))
    sums[...] += lax.dot_general(onehot, h, dn,
                                 preferred_element_type=jnp.float32)
    cnts[...] += lax.dot_general(onehot, jnp.ones((BLK, D), jnp.float32), dn,
                                 preferred_element_type=jnp.float32)

    @pl.when(i == pl.num_programs(0) - 1)
    def _():
        pooled = sums[...] / jnp.maximum(cnts[...], 1.0)
        out_ref[...] = jnp.dot(pooled, linw_ref[...],
                               preferred_element_type=jnp.float32) + linb_ref[...]


@jax.jit
def _final_call(ya, yb, xp, dis, b2d, batch_pad, linw_pad, linb_pad):
    return pl.pallas_call(
        _final_body,
        grid=(N_PAD // BLK,),
        in_specs=[
            pl.BlockSpec((BLK, D), lambda i: (i, 0)),
            pl.BlockSpec((BLK, D), lambda i: (i, 0)),
            pl.BlockSpec((BLK, D), lambda i: (i, 0)),
            pl.BlockSpec((BLK,), lambda i: (i,)),
            pl.BlockSpec((1, D), lambda i: (0, 0)),
            pl.BlockSpec((BLK,), lambda i: (i,)),
            pl.BlockSpec((D, D), lambda i: (0, 0)),
            pl.BlockSpec((1, D), lambda i: (0, 0)),
        ],
        out_specs=pl.BlockSpec((NUM_GRAPHS, D), lambda i: (0, 0)),
        out_shape=jax.ShapeDtypeStruct((NUM_GRAPHS, D), jnp.float32),
        scratch_shapes=[
            pltpu.VMEM((NUM_GRAPHS, D), jnp.float32),
            pltpu.VMEM((NUM_GRAPHS, D), jnp.float32),
        ],
    )(ya, yb, xp, dis, b2d, batch_pad, linw_pad, linb_pad)


def kernel(x, edge_index, batch, W1, b1, W2, b2, W3, b3, lin_W, lin_b):
    src = edge_index[0]
    dst = edge_index[1]
    pad_e = E_PAD - E
    src_p = jnp.concatenate(
        [src, jnp.zeros((pad_e,), jnp.int32)]).reshape(NTILES, CPT, CHUNK)
    dst_p = jnp.concatenate(
        [dst, jnp.full((pad_e,), N, jnp.int32)]).reshape(NTILES, CPT, CHUNK)
    zeros128 = jnp.zeros((RPT, D), jnp.float32)
    ones128 = jnp.ones((CHUNK, D), jnp.float32)

    degp = _deg_call(dst_p, zeros128, ones128)
    deg = degp[0, :N, 0] + degp[1, :N, 0] + 1.0
    deg_pad = jnp.concatenate([deg, jnp.zeros((N_PAD - N,), jnp.float32)])
    x_pad = jnp.concatenate(
        [x, jnp.zeros((N_PAD - N, D), jnp.float32)], axis=0)

    x1, dis = _dense1_call(x_pad, W1, deg_pad)
    y1 = _spmm_call(x1, src_p, dst_p, zeros128)
    x2 = _mid_call(y1[0], y1[1], x1, dis, b1.reshape(1, D), W2)
    y2 = _spmm_call(x2, src_p, dst_p, zeros128)
    x3 = _mid_call(y2[0], y2[1], x2, dis, b2.reshape(1, D), W3)
    y3 = _spmm_call(x3, src_p, dst_p, zeros128)

    batch_pad = jnp.concatenate(
        [batch, jnp.full((N_PAD - N,), NUM_GRAPHS + 7, jnp.int32)])
    linw_pad = jnp.pad(lin_W, ((0, 0), (0, D - lin_W.shape[1])))
    linb_pad = jnp.pad(lin_b, (0, D - lin_b.shape[0])).reshape(1, D)
    outf = _final_call(y3[0], y3[1], x3, dis, b3.reshape(1, D),
                       batch_pad, linw_pad, linb_pad)
    return outf[:, :1]


# speedup vs baseline: 7.4798x; 1.1190x over previous
import functools

import jax
import jax.numpy as jnp
from jax import lax
from jax.experimental import pallas as pl
from jax.experimental.pallas import tpu as pltpu
from jax.experimental.pallas import tpu_sc as plsc

N = 10000
E = 320000
D = 128
NUM_GRAPHS = 128

NTILES = 32
CHUNK = 128
CPT = 80
EPT = CHUNK * CPT
E_PAD = EPT * NTILES
N_PAD = 10240
RPT = N_PAD // 16

BLK = 512


def _mesh():
    return plsc.VectorSubcoreMesh(core_axis_name="c", subcore_axis_name="s")


def _deg_body(dst_hbm, zeros_hbm, ones_hbm, out_hbm, dst_v, ones_v, acc_sh):
    c = lax.axis_index("c")
    s = lax.axis_index("s")
    wid = c * 16 + s
    pltpu.sync_copy(zeros_hbm, acc_sh.at[pl.ds(s * RPT, RPT)])
    pltpu.sync_copy(dst_hbm.at[wid], dst_v)
    pltpu.sync_copy(ones_hbm, ones_v)
    plsc.subcore_barrier()

    def step(i, carry):
        pltpu.sync_copy(ones_v, acc_sh.at[dst_v.at[i]], add=True)
        return carry

    lax.fori_loop(0, CPT, step, 0)
    plsc.subcore_barrier()
    pltpu.sync_copy(acc_sh.at[pl.ds(s * RPT, RPT)],
                    out_hbm.at[c, pl.ds(s * RPT, RPT)])


@jax.jit
def _deg_call(dst_p, zeros128, ones128):
    return pl.kernel(
        _deg_body,
        out_type=jax.ShapeDtypeStruct((2, N_PAD, D), jnp.float32),
        mesh=_mesh(),
        scratch_types=[
            pltpu.VMEM((CPT, CHUNK), jnp.int32),
            pltpu.VMEM((CHUNK, D), jnp.float32),
            pltpu.VMEM_SHARED((N_PAD, D), jnp.float32),
        ],
    )(dst_p, zeros128, ones128)


def _spmm_body(x_hbm, src_hbm, dst_hbm, zeros_hbm, out_hbm,
               src_v, d0, d1, r0, r1, acc_sh, sd0, sd1, sg0, sg1):
    c = lax.axis_index("c")
    s = lax.axis_index("s")
    wid = c * 16 + s
    pltpu.async_copy(src_hbm.at[wid], src_v, sd0)
    pltpu.sync_copy(zeros_hbm, acc_sh.at[pl.ds(s * RPT, RPT)])
    pltpu.make_async_copy(src_hbm.at[wid], src_v, sd0).wait()
    plsc.subcore_barrier()

    pltpu.async_copy(dst_hbm.at[wid, pl.ds(0, 1)], d0, sd0)
    pltpu.async_copy(dst_hbm.at[wid, pl.ds(1, 1)], d1, sd1)
    pltpu.async_copy(x_hbm.at[src_v.at[0]], r0, sg0)

    def step(i2, carry):
        i = 2 * i2
        pltpu.async_copy(x_hbm.at[src_v.at[i + 1]], r1, sg1)
        pltpu.make_async_copy(x_hbm.at[src_v.at[i]], r0, sg0).wait()
        pltpu.make_async_copy(dst_hbm.at[wid, pl.ds(0, 1)], d0, sd0).wait()
        pltpu.sync_copy(r0, acc_sh.at[d0.at[0]], add=True)

        @pl.when(i2 < CPT // 2 - 1)
        def _():
            pltpu.async_copy(x_hbm.at[src_v.at[i + 2]], r0, sg0)
            pltpu.async_copy(dst_hbm.at[wid, pl.ds(i + 2, 1)], d0, sd0)

        pltpu.make_async_copy(x_hbm.at[src_v.at[i + 1]], r1, sg1).wait()
        pltpu.make_async_copy(dst_hbm.at[wid, pl.ds(1, 1)], d1, sd1).wait()
        pltpu.sync_copy(r1, acc_sh.at[d1.at[0]], add=True)

        @pl.when(i2 < CPT // 2 - 1)
        def _():
            pltpu.async_copy(dst_hbm.at[wid, pl.ds(i + 3, 1)], d1, sd1)

        return carry

    lax.fori_loop(0, CPT // 2, step, 0)
    plsc.subcore_barrier()
    pltpu.sync_copy(acc_sh.at[pl.ds(s * RPT, RPT)],
                    out_hbm.at[c, pl.ds(s * RPT, RPT)])


@jax.jit
def _spmm_call(x_pad, src_p, dst_p, zeros128):
    return pl.kernel(
        _spmm_body,
        out_type=jax.ShapeDtypeStruct((2, N_PAD, D), jnp.float32),
        mesh=_mesh(),
        scratch_types=[
            pltpu.VMEM((CPT, CHUNK), jnp.int32),
            pltpu.VMEM((1, CHUNK), jnp.int32),
            pltpu.VMEM((1, CHUNK), jnp.int32),
            pltpu.VMEM((CHUNK, D), jnp.float32),
            pltpu.VMEM((CHUNK, D), jnp.float32),
            pltpu.VMEM_SHARED((N_PAD, D), jnp.float32),
            pltpu.SemaphoreType.DMA,
            pltpu.SemaphoreType.DMA,
            pltpu.SemaphoreType.DMA,
            pltpu.SemaphoreType.DMA,
        ],
    )(x_pad, src_p, dst_p, zeros128)


def _dense1_body(x_ref, w_ref, deg_ref, xs_ref, dis_ref):
    deg = deg_ref[...]
    dis = jnp.where(deg > 0, lax.rsqrt(deg), 0.0)
    h = jnp.dot(x_ref[...], w_ref[...], preferred_element_type=jnp.float32)
    xs_ref[...] = h * dis[:, None]
    dis_ref[...] = dis


@jax.jit
def _dense1_call(x_pad, W1, deg_pad):
    return pl.pallas_call(
        _dense1_body,
        grid=(N_PAD // BLK,),
        in_specs=[
            pl.BlockSpec((BLK, D), lambda i: (i, 0)),
            pl.BlockSpec((D, D), lambda i: (0, 0)),
            pl.BlockSpec((BLK,), lambda i: (i,)),
        ],
        out_specs=[
            pl.BlockSpec((BLK, D), lambda i: (i, 0)),
            pl.BlockSpec((BLK,), lambda i: (i,)),
        ],
        out_shape=[
            jax.ShapeDtypeStruct((N_PAD, D), jnp.float32),
            jax.ShapeDtypeStruct((N_PAD,), jnp.float32),
        ],
    )(x_pad, W1, deg_pad)


def _mid_body(ya_ref, yb_ref, xp_ref, dis_ref, b_ref, w_ref, out_ref):
    dis = dis_ref[...]
    t = (ya_ref[...] + yb_ref[...] + xp_ref[...]) * dis[:, None] + b_ref[...]
    h = jnp.maximum(t, 0.0)
    out_ref[...] = jnp.dot(h, w_ref[...],
                           preferred_element_type=jnp.float32) * dis[:, None]


@jax.jit
def _mid_call(ya, yb, xp, dis, b2d, W):
    return pl.pallas_call(
        _mid_body,
        grid=(N_PAD // BLK,),
        in_specs=[
            pl.BlockSpec((BLK, D), lambda i: (i, 0)),
            pl.BlockSpec((BLK, D), lambda i: (i, 0)),
            pl.BlockSpec((BLK, D), lambda i: (i, 0)),
            pl.BlockSpec((BLK,), lambda i: (i,)),
            pl.BlockSpec((1, D), lambda i: (0, 0)),
            pl.BlockSpec((D, D), lambda i: (0, 0)),
        ],
        out_specs=pl.BlockSpec((BLK, D), lambda i: (i, 0)),
        out_shape=jax.ShapeDtypeStruct((N_PAD, D), jnp.float32),
    )(ya, yb, xp, dis, b2d, W)


def _final_body(ya_ref, yb_ref, xp_ref, dis_ref, b_ref, batch_ref,
                linw_ref, linb_ref, out_ref, sums, cnts):
    i = pl.program_id(0)

    @pl.when(i == 0)
    def _():
        sums[...] = jnp.zeros_like(sums)
        cnts[...] = jnp.zeros_like(cnts)

    dis = dis_ref[...]
    t = (ya_ref[...] + yb_ref[...] + xp_ref[...]) * dis[:, None] + b_ref[...]
    h = jnp.maximum(t, 0.0)
    bt = batch_ref[...]
    onehot = (lax.broadcasted_iota(jnp.int32, (BLK, NUM_GRAPHS), 1)
              == bt[:, None]).astype(jnp.float32)
    dn = (((0,), (0,)), ((), ()))
    sums[...] += lax.dot_general(onehot, h, dn,
                                 preferred_element_type=jnp.float32,
                                 precision=lax.Precision.HIGHEST)
    cnts[...] += lax.dot_general(onehot, jnp.ones((BLK, D), jnp.float32), dn,
                                 preferred_element_type=jnp.float32,
                                 precision=lax.Precision.HIGHEST)

    @pl.when(i == pl.num_programs(0) - 1)
    def _():
        pooled = sums[...] / jnp.maximum(cnts[...], 1.0)
        out_ref[...] = jnp.dot(pooled, linw_ref[...],
                               preferred_element_type=jnp.float32) + linb_ref[...]


@jax.jit
def _final_call(ya, yb, xp, dis, b2d, batch_pad, linw_pad, linb_pad):
    return pl.pallas_call(
        _final_body,
        grid=(N_PAD // BLK,),
        in_specs=[
            pl.BlockSpec((BLK, D), lambda i: (i, 0)),
            pl.BlockSpec((BLK, D), lambda i: (i, 0)),
            pl.BlockSpec((BLK, D), lambda i: (i, 0)),
            pl.BlockSpec((BLK,), lambda i: (i,)),
            pl.BlockSpec((1, D), lambda i: (0, 0)),
            pl.BlockSpec((BLK,), lambda i: (i,)),
            pl.BlockSpec((D, D), lambda i: (0, 0)),
            pl.BlockSpec((1, D), lambda i: (0, 0)),
        ],
        out_specs=pl.BlockSpec((NUM_GRAPHS, D), lambda i: (0, 0)),
        out_shape=jax.ShapeDtypeStruct((NUM_GRAPHS, D), jnp.float32),
        scratch_shapes=[
            pltpu.VMEM((NUM_GRAPHS, D), jnp.float32),
            pltpu.VMEM((NUM_GRAPHS, D), jnp.float32),
        ],
    )(ya, yb, xp, dis, b2d, batch_pad, linw_pad, linb_pad)


def kernel(x, edge_index, batch, W1, b1, W2, b2, W3, b3, lin_W, lin_b):
    src = edge_index[0]
    dst = edge_index[1]
    pad_e = E_PAD - E
    src_p = jnp.concatenate(
        [src, jnp.zeros((pad_e,), jnp.int32)]).reshape(NTILES, CPT, CHUNK)
    dst_p = jnp.concatenate(
        [dst, jnp.full((pad_e,), N, jnp.int32)]).reshape(NTILES, CPT, CHUNK)
    zeros128 = jnp.zeros((RPT, D), jnp.float32)
    ones128 = jnp.ones((CHUNK, D), jnp.float32)

    degp = _deg_call(dst_p, zeros128, ones128)
    deg = degp[0, :N, 0] + degp[1, :N, 0] + 1.0
    deg_pad = jnp.concatenate([deg, jnp.zeros((N_PAD - N,), jnp.float32)])
    x_pad = jnp.concatenate(
        [x, jnp.zeros((N_PAD - N, D), jnp.float32)], axis=0)

    x1, dis = _dense1_call(x_pad, W1, deg_pad)
    y1 = _spmm_call(x1, src_p, dst_p, zeros128)
    x2 = _mid_call(y1[0], y1[1], x1, dis, b1.reshape(1, D), W2)
    y2 = _spmm_call(x2, src_p, dst_p, zeros128)
    x3 = _mid_call(y2[0], y2[1], x2, dis, b2.reshape(1, D), W3)
    y3 = _spmm_call(x3, src_p, dst_p, zeros128)

    batch_pad = jnp.concatenate(
        [batch, jnp.full((N_PAD - N,), NUM_GRAPHS + 7, jnp.int32)])
    linw_pad = jnp.pad(lin_W, ((0, 0), (0, D - lin_W.shape[1])))
    linb_pad = jnp.pad(lin_b, (0, D - lin_b.shape[0])).reshape(1, D)
    outf = _final_call(y3[0], y3[1], x3, dis, b3.reshape(1, D),
                       batch_pad, linw_pad, linb_pad)
    return outf[:, :1]
